# baseline (speedup 1.0000x reference)
import jax
import jax.numpy as jnp
from jax import lax
from jax.experimental import pallas as pl
from jax.experimental.pallas import tpu as pltpu
from jax.experimental.pallas import tpu_sc as plsc

B, S, D = 8, 2048, 512
NC, NS, L = 2, 16, 16
NW = NC * NS
TOK_W = (B * S) // NW
CH_ROW = S // TOK_W
SUB = 32
NSUB = TOK_W // SUB
VPT = SUB // L
NBUF = 3


def _pe_add_body(seqs_hbm, masks_hbm, pe_hbm, out_hbm,
                 mask_v, delta_v, seq_a, seq_b, seq_c, pe_a, pe_b, pe_c,
                 sem_sa, sem_sb, sem_sc, sem_pa, sem_pb, sem_pc,
                 sem_oa, sem_ob, sem_oc):
    wid = lax.axis_index("s") * NC + lax.axis_index("c")
    row = wid // CH_ROW
    chunk = lax.rem(wid, CH_ROW)

    seq_bufs = (seq_a, seq_b, seq_c)
    pe_bufs = (pe_a, pe_b, pe_c)
    sem_seq = (sem_sa, sem_sb, sem_sc)
    sem_pe = (sem_pa, sem_pb, sem_pc)
    sem_out = (sem_oa, sem_ob, sem_oc)

    tok0 = wid * TOK_W

    def start_seq(sblk):
        slot = sblk % NBUF
        t0 = tok0 + sblk * SUB
        pltpu.async_copy(seqs_hbm.at[pl.ds(t0, SUB)], seq_bufs[slot],
                         sem_seq[slot])

    pltpu.sync_copy(masks_hbm.at[row], mask_v)
    for sblk in range(NBUF - 1):
        start_seq(sblk)

    def prefix_step(k, run):
        c = plsc.cumsum(mask_v[pl.ds(k * L, L)])
        return run + c[L - 1]

    run = lax.fori_loop(0, chunk * (TOK_W // L), prefix_step, jnp.int32(0))

    k0 = chunk * (TOK_W // L)
    fs_list = []
    for s in range(NSUB):
        fs_list.append(run)
        for k2 in range(VPT):
            k = k0 + s * VPT + k2
            m = mask_v[pl.ds(k * L, L)]
            c = plsc.cumsum(m)
            delta = c - 1 + (run - fs_list[s])
            delta_v[pl.ds(s * SUB + k2 * L, L)] = jnp.where(m > 0, delta, SUB)
            run = run + c[L - 1]

    fs_list.append(run)

    def start_pe(sblk):
        slot = sblk % NBUF
        cnt = fs_list[sblk + 1] - fs_list[sblk]
        for nr in (8, 16, 24, 32):
            @pl.when(jnp.logical_and(cnt > nr - 8, cnt <= nr))
            def _():
                pltpu.async_copy(
                    pe_hbm.at[pl.ds(fs_list[sblk] * D, nr * D)],
                    pe_bufs[slot].at[pl.ds(0, nr * D)], sem_pe[slot])

    def wait_in(sblk):
        slot = sblk % NBUF
        t0 = tok0 + sblk * SUB
        pltpu.make_async_copy(seqs_hbm.at[pl.ds(t0, SUB)], seq_bufs[slot],
                              sem_seq[slot]).wait()
        cnt = fs_list[sblk + 1] - fs_list[sblk]
        for nr in (8, 16, 24, 32):
            @pl.when(jnp.logical_and(cnt > nr - 8, cnt <= nr))
            def _():
                pltpu.make_async_copy(
                    pe_hbm.at[pl.ds(fs_list[sblk] * D, nr * D)],
                    pe_bufs[slot].at[pl.ds(0, nr * D)], sem_pe[slot]).wait()

    def wait_out(sblk):
        slot = sblk % NBUF
        t0 = tok0 + sblk * SUB
        pltpu.make_async_copy(seq_bufs[slot], out_hbm.at[pl.ds(t0, SUB)],
                              sem_out[slot]).wait()

    for sblk in range(NBUF - 1):
        start_pe(sblk)

    for sblk in range(NSUB):
        slot = sblk % NBUF
        t0 = tok0 + sblk * SUB
        if sblk + NBUF - 1 < NSUB:
            if sblk >= 1:
                wait_out(sblk - 1)
            start_seq(sblk + NBUF - 1)
            start_pe(sblk + NBUF - 1)
        wait_in(sblk)

        @pl.loop(0, SUB)
        def _add(t):
            d = delta_v[pl.ds(sblk * SUB + t, L)][0]

            @pl.when(d < SUB)
            def _():
                for dd in range(D // L):
                    v = pe_bufs[slot][pl.ds(d * D + dd * L, L)]
                    plsc.addupdate(seq_bufs[slot].at[t, pl.ds(dd * L, L)], v)

        pltpu.async_copy(seq_bufs[slot], out_hbm.at[pl.ds(t0, SUB)],
                         sem_out[slot])

    for sblk in range(NSUB - NBUF, NSUB):
        wait_out(sblk)


def kernel(seqs, masks, pe):
    seqs_flat = seqs.reshape(B * S, D)
    masks_i = masks.astype(jnp.int32)
    mesh = plsc.VectorSubcoreMesh(core_axis_name="c", subcore_axis_name="s")
    out = pl.kernel(
        _pe_add_body,
        out_type=jax.ShapeDtypeStruct((B * S, D), jnp.float32),
        mesh=mesh,
        compiler_params=pltpu.CompilerParams(needs_layout_passes=False),
        scratch_types=[
            pltpu.VMEM((S,), jnp.int32),
            pltpu.VMEM((TOK_W + L,), jnp.int32),
            pltpu.VMEM((SUB, D), jnp.float32),
            pltpu.VMEM((SUB, D), jnp.float32),
            pltpu.VMEM((SUB, D), jnp.float32),
            pltpu.VMEM((SUB * D,), jnp.float32),
            pltpu.VMEM((SUB * D,), jnp.float32),
            pltpu.VMEM((SUB * D,), jnp.float32),
            pltpu.SemaphoreType.DMA,
            pltpu.SemaphoreType.DMA,
            pltpu.SemaphoreType.DMA,
            pltpu.SemaphoreType.DMA,
            pltpu.SemaphoreType.DMA,
            pltpu.SemaphoreType.DMA,
            pltpu.SemaphoreType.DMA,
            pltpu.SemaphoreType.DMA,
            pltpu.SemaphoreType.DMA,
        ],
    )(seqs_flat, masks_i, pe.reshape(S * D))
    return out.reshape(B, S, D)


# speedup vs baseline: 1.0127x; 1.0127x over previous
import jax
import jax.numpy as jnp
from jax import lax
from jax.experimental import pallas as pl
from jax.experimental.pallas import tpu as pltpu
from jax.experimental.pallas import tpu_sc as plsc

B, S, D = 8, 2048, 512
NC, NS, L = 2, 16, 16
NW = NC * NS
TOK_W = (B * S) // NW
CH_ROW = S // TOK_W
SUB = 64
NSUB = TOK_W // SUB
WINT = 32
NWIN = TOK_W // WINT
WVPT = WINT // L


def _pe_add_body(seqs_hbm, masks_hbm, pe_hbm, out_hbm,
                 mask_v, delta_v, seq_a, seq_b, pe_a, pe_b,
                 sem_sa, sem_sb, sem_pa, sem_pb, sem_oa, sem_ob):
    wid = lax.axis_index("s") * NC + lax.axis_index("c")
    row = wid // CH_ROW
    chunk = lax.rem(wid, CH_ROW)

    seq_bufs = (seq_a, seq_b)
    pe_bufs = (pe_a, pe_b)
    sem_seq = (sem_sa, sem_sb)
    sem_pe = (sem_pa, sem_pb)
    sem_out = (sem_oa, sem_ob)

    tok0 = wid * TOK_W

    def start_seq(sblk):
        slot = sblk % 2
        t0 = tok0 + sblk * SUB
        pltpu.async_copy(seqs_hbm.at[pl.ds(t0, SUB)], seq_bufs[slot],
                         sem_seq[slot])

    pltpu.sync_copy(masks_hbm.at[row], mask_v)
    start_seq(0)

    def prefix_step(k, run):
        c = plsc.cumsum(mask_v[pl.ds(k * L, L)])
        return run + c[L - 1]

    run = lax.fori_loop(0, chunk * (TOK_W // L), prefix_step, jnp.int32(0))

    k0 = chunk * (TOK_W // L)
    fs_list = []
    for w in range(NWIN):
        fs_list.append(run)
        for k2 in range(WVPT):
            k = k0 + w * WVPT + k2
            m = mask_v[pl.ds(k * L, L)]
            c = plsc.cumsum(m)
            delta = c - 1 + (run - fs_list[w])
            delta_v[pl.ds(w * WINT + k2 * L, L)] = jnp.where(m > 0, delta, WINT)
            run = run + c[L - 1]

    def start_pe(w):
        slot = w % 2
        pltpu.async_copy(pe_hbm.at[pl.ds(fs_list[w] * D, WINT * D)],
                         pe_bufs[slot], sem_pe[slot])

    def wait_pe(w):
        slot = w % 2
        pltpu.make_async_copy(pe_hbm.at[pl.ds(fs_list[w] * D, WINT * D)],
                              pe_bufs[slot], sem_pe[slot]).wait()

    def wait_seq_in(sblk):
        slot = sblk % 2
        t0 = tok0 + sblk * SUB
        pltpu.make_async_copy(seqs_hbm.at[pl.ds(t0, SUB)], seq_bufs[slot],
                              sem_seq[slot]).wait()

    def wait_out(sblk):
        slot = sblk % 2
        t0 = tok0 + sblk * SUB
        pltpu.make_async_copy(seq_bufs[slot], out_hbm.at[pl.ds(t0, SUB)],
                              sem_out[slot]).wait()

    start_pe(0)
    start_pe(1)

    for sblk in range(NSUB):
        slot = sblk % 2
        t0 = tok0 + sblk * SUB
        if sblk + 1 < NSUB:
            if sblk >= 1:
                wait_out(sblk - 1)
            start_seq(sblk + 1)
        wait_seq_in(sblk)

        for half in range(2):
            w = 2 * sblk + half
            wslot = w % 2
            wait_pe(w)

            @pl.loop(half * WINT, (half + 1) * WINT)
            def _add(t):
                d = delta_v[pl.ds(sblk * SUB + t, L)][0]

                @pl.when(d < WINT)
                def _():
                    for dd in range(D // L):
                        v = pe_bufs[wslot][pl.ds(d * D + dd * L, L)]
                        plsc.addupdate(seq_bufs[slot].at[t, pl.ds(dd * L, L)],
                                       v)

            if w + 2 < NWIN:
                start_pe(w + 2)

        pltpu.async_copy(seq_bufs[slot], out_hbm.at[pl.ds(t0, SUB)],
                         sem_out[slot])

    for sblk in (NSUB - 2, NSUB - 1):
        wait_out(sblk)


def kernel(seqs, masks, pe):
    seqs_flat = seqs.reshape(B * S, D)
    masks_i = masks.astype(jnp.int32)
    mesh = plsc.VectorSubcoreMesh(core_axis_name="c", subcore_axis_name="s")
    out = pl.kernel(
        _pe_add_body,
        out_type=jax.ShapeDtypeStruct((B * S, D), jnp.float32),
        mesh=mesh,
        compiler_params=pltpu.CompilerParams(needs_layout_passes=False),
        scratch_types=[
            pltpu.VMEM((S,), jnp.int32),
            pltpu.VMEM((TOK_W + L,), jnp.int32),
            pltpu.VMEM((SUB, D), jnp.float32),
            pltpu.VMEM((SUB, D), jnp.float32),
            pltpu.VMEM((WINT * D,), jnp.float32),
            pltpu.VMEM((WINT * D,), jnp.float32),
            pltpu.SemaphoreType.DMA,
            pltpu.SemaphoreType.DMA,
            pltpu.SemaphoreType.DMA,
            pltpu.SemaphoreType.DMA,
            pltpu.SemaphoreType.DMA,
            pltpu.SemaphoreType.DMA,
        ],
    )(seqs_flat, masks_i, pe.reshape(S * D))
    return out.reshape(B, S, D)
